# baseline (speedup 1.0000x reference)
import functools

import jax
import jax.numpy as jnp
from jax import lax
from jax.experimental import pallas as pl
from jax.experimental.pallas import tpu as pltpu
from jax.experimental.pallas import tpu_sc as plsc

B, L = 4096, 50
DIM = 32
LANES = 16
VOCAB = 1000000

NC, NS = 2, 16
NW = NC * NS
RPW = B // NW
ROWS_PER_CHUNK = 2
G = ROWS_PER_CHUNK * L
CHUNKS = RPW // ROWS_PER_CHUNK
LPAD = 64
PACK = 128 // DIM

BLK = 768
FULL_COLS = (VOCAB // 128) * 128
N_SUPER = FULL_COLS // BLK
BLOCKS_PER_W = (N_SUPER + NW - 1) // NW
OUT_PER_BLK = BLK // PACK
IN_STRIDE = BLK + 8
PACKED_ROWS = VOCAB // PACK + 16


def _permute(vec, idx):
    return lax.gather(
        vec, idx[:, None],
        dimension_numbers=lax.GatherDimensionNumbers(
            offset_dims=(), collapsed_slice_dims=(0,), start_index_map=(0,)),
        slice_sizes=(1,),
        mode=lax.GatherScatterMode.PROMISE_IN_BOUNDS)


def _bcast(vec, j):
    return _permute(vec, jnp.full((LANES,), j, jnp.int32))


def _allsum(vec):
    idx = jnp.arange(LANES, dtype=jnp.int32)
    for sh in (1, 2, 4, 8):
        vec = vec + _permute(vec, idx ^ sh)
    return vec


def _transpose_block(in_ref, out_ref, ncols):
    iota = lax.iota(jnp.int32, LANES)
    hi = iota + LANES

    @plsc.parallel_loop(0, ncols // PACK, unroll=4)
    def _(q):
        for m in range(PACK):
            col = jnp.full((LANES,), q * PACK + m, jnp.int32)
            g0 = plsc.load_gather(in_ref, [iota, col])
            g1 = plsc.load_gather(in_ref, [hi, col])
            out_ref[q, pl.ds(m * DIM, LANES)] = g0
            out_ref[q, pl.ds(m * DIM + LANES, LANES)] = g1


def _make_transpose_kernel():
    mesh = plsc.VectorSubcoreMesh(core_axis_name="c", subcore_axis_name="s")

    @functools.partial(
        pl.kernel,
        mesh=mesh,
        out_type=jax.ShapeDtypeStruct((PACKED_ROWS, 128), jnp.float32),
        compiler_params=pltpu.CompilerParams(needs_layout_passes=False),
        scratch_types=[
            pltpu.VMEM((DIM, IN_STRIDE), jnp.float32),
            pltpu.VMEM((DIM, IN_STRIDE), jnp.float32),
            pltpu.VMEM((OUT_PER_BLK, 128), jnp.float32),
            pltpu.VMEM((OUT_PER_BLK, 128), jnp.float32),
            pltpu.SemaphoreType.DMA,
            pltpu.SemaphoreType.DMA,
            pltpu.SemaphoreType.DMA,
            pltpu.SemaphoreType.DMA,
        ],
    )
    def tr(wvt_hbm, tail_hbm, out_hbm,
           in_a, in_b, out_a, out_b, sin_a, sin_b, sout_a, sout_b):
        wid = lax.axis_index("s") * NC + lax.axis_index("c")
        base = wid * BLOCKS_PER_W

        def blk_of(i):
            return jnp.minimum(base + i, N_SUPER - 1)

        def start_in(i, buf, sem):
            c0 = blk_of(i) * BLK
            pltpu.async_copy(wvt_hbm.at[:, pl.ds(c0, BLK)],
                             buf.at[:, pl.ds(0, BLK)], sem)

        def wait_in(i, buf, sem):
            c0 = blk_of(i) * BLK
            pltpu.make_async_copy(
                wvt_hbm.at[:, pl.ds(c0, BLK)],
                buf.at[:, pl.ds(0, BLK)], sem).wait()

        def start_out(i, buf, sem):
            r0 = blk_of(i) * OUT_PER_BLK
            pltpu.async_copy(buf, out_hbm.at[pl.ds(r0, OUT_PER_BLK)], sem)

        def wait_out(i, buf, sem):
            r0 = blk_of(i) * OUT_PER_BLK
            pltpu.make_async_copy(
                buf, out_hbm.at[pl.ds(r0, OUT_PER_BLK)], sem).wait()

        start_in(0, in_a, sin_a)

        def body(t, carry):
            i = 2 * t
            wait_in(i, in_a, sin_a)
            start_in(i + 1, in_b, sin_b)

            @pl.when(t > 0)
            def _():
                wait_out(i - 2, out_a, sout_a)

            _transpose_block(in_a, out_a, BLK)
            start_out(i, out_a, sout_a)

            wait_in(i + 1, in_b, sin_b)

            @pl.when(i + 2 < BLOCKS_PER_W)
            def _():
                start_in(i + 2, in_a, sin_a)

            @pl.when(t > 0)
            def _():
                wait_out(i - 1, out_b, sout_b)

            _transpose_block(in_b, out_b, BLK)
            start_out(i + 1, out_b, sout_b)
            return carry

        lax.fori_loop(0, BLOCKS_PER_W // 2, body, 0)

        i_last = BLOCKS_PER_W - 1
        wait_out(i_last - 2, out_a, sout_a)
        wait_in(i_last, in_a, sin_a)
        _transpose_block(in_a, out_a, BLK)
        start_out(i_last, out_a, sout_a)
        wait_out(i_last - 1, out_b, sout_b)
        wait_out(i_last, out_a, sout_a)

        @pl.when(wid == 0)
        def _():
            pltpu.sync_copy(tail_hbm, in_b.at[:, pl.ds(0, 128)])
            _transpose_block(in_b, out_b, 128)
            pltpu.sync_copy(
                out_b.at[pl.ds(0, 32)],
                out_hbm.at[pl.ds(FULL_COLS // PACK, 32)])

    return tr


def _make_emb_kernel():
    mesh = plsc.VectorSubcoreMesh(core_axis_name="c", subcore_axis_name="s")

    @functools.partial(
        pl.kernel,
        mesh=mesh,
        out_type=jax.ShapeDtypeStruct((B, DIM), jnp.float32),
        compiler_params=pltpu.CompilerParams(needs_layout_passes=False),
        scratch_types=[
            pltpu.VMEM((CHUNKS + 1, G), jnp.int32),
            pltpu.VMEM((RPW, LPAD), jnp.float32),
            pltpu.VMEM((RPW, LPAD), jnp.int32),
            pltpu.VMEM((G, 4 * DIM), jnp.float32),
            pltpu.VMEM((G, 4 * DIM), jnp.float32),
            pltpu.VMEM((RPW, DIM), jnp.float32),
            pltpu.SemaphoreType.DMA,
            pltpu.SemaphoreType.DMA,
        ],
    )
    def emb(idx_hbm, w_hbm, off_hbm, table_hbm, out_hbm,
            idx_v, w_v, off_v, buf_a, buf_b, out_v, sem_a, sem_b):
        wid = lax.axis_index("s") * NC + lax.axis_index("c")
        pltpu.sync_copy(idx_hbm.at[wid], idx_v)
        pltpu.sync_copy(w_hbm.at[wid], w_v)
        pltpu.sync_copy(off_hbm.at[wid], off_v)

        def start(jj, buf, sem):
            pltpu.async_copy(table_hbm.at[idx_v.at[jj]], buf, sem)

        def wait(jj, buf, sem):
            pltpu.make_async_copy(table_hbm.at[idx_v.at[jj]], buf, sem).wait()

        def compute(buf, jj):
            for r in range(ROWS_PER_CHUNK):
                b = ROWS_PER_CHUNK * jj + r
                wblk = [w_v[b, pl.ds(k * LANES, LANES)] for k in range(4)]
                oblk = [off_v[b, pl.ds(k * LANES, LANES)] for k in range(4)]
                dv = wblk[0] + wblk[1] + wblk[2] + wblk[3]
                inv = 1.0 / _allsum(dv)
                a0 = jnp.zeros((LANES,), jnp.float32)
                a1 = jnp.zeros((LANES,), jnp.float32)
                for l in range(L):
                    w = _bcast(wblk[l // LANES], l % LANES)
                    o = oblk[l // LANES][l % LANES]
                    pos = r * L + l
                    a0 = a0 + w * buf[pos, pl.ds(o, LANES)]
                    a1 = a1 + w * buf[pos, pl.ds(o + LANES, LANES)]
                out_v[b, pl.ds(0, LANES)] = a0 * inv
                out_v[b, pl.ds(LANES, LANES)] = a1 * inv

        start(0, buf_a, sem_a)

        def body(j2, carry):
            jj = 2 * j2
            wait(jj, buf_a, sem_a)
            start(jj + 1, buf_b, sem_b)
            compute(buf_a, jj)
            wait(jj + 1, buf_b, sem_b)
            start(jj + 2, buf_a, sem_a)
            compute(buf_b, jj + 1)
            return carry

        lax.fori_loop(0, CHUNKS // 2, body, 0)
        wait(CHUNKS, buf_a, sem_a)
        pltpu.sync_copy(out_v, out_hbm.at[pl.ds(wid * RPW, RPW)])

    return emb


_tr_kernel = _make_transpose_kernel()
_emb_kernel = _make_emb_kernel()


def kernel(x_id, x_weight, word_vector):
    wv = word_vector.astype(jnp.float32)
    wvt = wv.T
    n_tail = VOCAB - FULL_COLS
    tail = jnp.pad(wv[FULL_COLS:], ((0, 128 - n_tail), (0, 0))).T
    table = _tr_kernel(wvt, tail)

    ids = x_id.astype(jnp.int32)
    idx = (ids // PACK).reshape(NW, CHUNKS, G)
    idx = jnp.concatenate([idx, idx[:, :1]], axis=1)
    off = (ids % PACK) * DIM
    off = jnp.pad(off, ((0, 0), (0, LPAD - L))).reshape(NW, RPW, LPAD)
    w = jnp.pad(x_weight.astype(jnp.float32), ((0, 0), (0, LPAD - L)))
    w = w.reshape(NW, RPW, LPAD)
    return _emb_kernel(idx, w, off, table)


# speedup vs baseline: 2.6272x; 2.6272x over previous
import functools

import jax
import jax.numpy as jnp
from jax import lax
from jax.experimental import pallas as pl
from jax.experimental.pallas import tpu as pltpu
from jax.experimental.pallas import tpu_sc as plsc

B, L = 4096, 50
DIM = 32
LANES = 16
VOCAB = 1000000

NC, NS = 2, 16
NW = NC * NS
RPW = B // NW
ROWS_PER_CHUNK = 2
G = ROWS_PER_CHUNK * L
CHUNKS = RPW // ROWS_PER_CHUNK
LPAD = 64
PACK = 128 // DIM

BLK = 768
FULL_COLS = (VOCAB // 128) * 128
N_SUPER = FULL_COLS // BLK
BLOCKS_PER_W = (N_SUPER + NW - 1) // NW
OUT_PER_BLK = BLK // PACK
IN_STRIDE = BLK + 8
PACKED_ROWS = VOCAB // PACK + 16


def _permute(vec, idx):
    return lax.gather(
        vec, idx[:, None],
        dimension_numbers=lax.GatherDimensionNumbers(
            offset_dims=(), collapsed_slice_dims=(0,), start_index_map=(0,)),
        slice_sizes=(1,),
        mode=lax.GatherScatterMode.PROMISE_IN_BOUNDS)


def _bcast(vec, j):
    return _permute(vec, jnp.full((LANES,), j, jnp.int32))


def _allsum(vec):
    idx = jnp.arange(LANES, dtype=jnp.int32)
    for sh in (1, 2, 4, 8):
        vec = vec + _permute(vec, idx ^ sh)
    return vec


def _transpose_block(in_ref, out_ref, ncols):
    iota = lax.iota(jnp.int32, LANES)
    perm_idx = {s: iota ^ s for s in (1, 2, 4, 8)}
    masks = {s: (iota & s) == 0 for s in (1, 2, 4, 8)}

    @plsc.parallel_loop(0, ncols // LANES, unroll=2)
    def _(g):
        v0 = g * LANES
        for h in range(2):
            t = [in_ref[16 * h + i, pl.ds(v0, LANES)] for i in range(LANES)]
            for s in (8, 4, 2, 1):
                for i in range(LANES):
                    if i & s:
                        continue
                    a, b = t[i], t[i | s]
                    pa = _permute(a, perm_idx[s])
                    pb = _permute(b, perm_idx[s])
                    t[i] = jnp.where(masks[s], a, pb)
                    t[i | s] = jnp.where(masks[s], pa, b)
            for j in range(LANES):
                out_ref[PACK * g + j // PACK,
                        pl.ds((j % PACK) * DIM + 16 * h, LANES)] = t[j]


def _make_transpose_kernel():
    mesh = plsc.VectorSubcoreMesh(core_axis_name="c", subcore_axis_name="s")

    @functools.partial(
        pl.kernel,
        mesh=mesh,
        out_type=jax.ShapeDtypeStruct((PACKED_ROWS, 128), jnp.float32),
        compiler_params=pltpu.CompilerParams(needs_layout_passes=False),
        scratch_types=[
            pltpu.VMEM((DIM, IN_STRIDE), jnp.float32),
            pltpu.VMEM((DIM, IN_STRIDE), jnp.float32),
            pltpu.VMEM((OUT_PER_BLK, 128), jnp.float32),
            pltpu.VMEM((OUT_PER_BLK, 128), jnp.float32),
            pltpu.SemaphoreType.DMA,
            pltpu.SemaphoreType.DMA,
            pltpu.SemaphoreType.DMA,
            pltpu.SemaphoreType.DMA,
        ],
    )
    def tr(wvt_hbm, tail_hbm, out_hbm,
           in_a, in_b, out_a, out_b, sin_a, sin_b, sout_a, sout_b):
        wid = lax.axis_index("s") * NC + lax.axis_index("c")
        base = wid * BLOCKS_PER_W

        def blk_of(i):
            return jnp.minimum(base + i, N_SUPER - 1)

        def start_in(i, buf, sem):
            c0 = blk_of(i) * BLK
            pltpu.async_copy(wvt_hbm.at[:, pl.ds(c0, BLK)],
                             buf.at[:, pl.ds(0, BLK)], sem)

        def wait_in(i, buf, sem):
            c0 = blk_of(i) * BLK
            pltpu.make_async_copy(
                wvt_hbm.at[:, pl.ds(c0, BLK)],
                buf.at[:, pl.ds(0, BLK)], sem).wait()

        def start_out(i, buf, sem):
            r0 = blk_of(i) * OUT_PER_BLK
            pltpu.async_copy(buf, out_hbm.at[pl.ds(r0, OUT_PER_BLK)], sem)

        def wait_out(i, buf, sem):
            r0 = blk_of(i) * OUT_PER_BLK
            pltpu.make_async_copy(
                buf, out_hbm.at[pl.ds(r0, OUT_PER_BLK)], sem).wait()

        start_in(0, in_a, sin_a)

        def body(t, carry):
            i = 2 * t
            wait_in(i, in_a, sin_a)
            start_in(i + 1, in_b, sin_b)

            @pl.when(t > 0)
            def _():
                wait_out(i - 2, out_a, sout_a)

            _transpose_block(in_a, out_a, BLK)
            start_out(i, out_a, sout_a)

            wait_in(i + 1, in_b, sin_b)

            @pl.when(i + 2 < BLOCKS_PER_W)
            def _():
                start_in(i + 2, in_a, sin_a)

            @pl.when(t > 0)
            def _():
                wait_out(i - 1, out_b, sout_b)

            _transpose_block(in_b, out_b, BLK)
            start_out(i + 1, out_b, sout_b)
            return carry

        lax.fori_loop(0, BLOCKS_PER_W // 2, body, 0)

        i_last = BLOCKS_PER_W - 1
        wait_out(i_last - 2, out_a, sout_a)
        wait_in(i_last, in_a, sin_a)
        _transpose_block(in_a, out_a, BLK)
        start_out(i_last, out_a, sout_a)
        wait_out(i_last - 1, out_b, sout_b)
        wait_out(i_last, out_a, sout_a)

        @pl.when(wid == 0)
        def _():
            pltpu.sync_copy(tail_hbm, in_b.at[:, pl.ds(0, 128)])
            _transpose_block(in_b, out_b, 128)
            pltpu.sync_copy(
                out_b.at[pl.ds(0, 32)],
                out_hbm.at[pl.ds(FULL_COLS // PACK, 32)])

    return tr


def _make_emb_kernel():
    mesh = plsc.VectorSubcoreMesh(core_axis_name="c", subcore_axis_name="s")

    @functools.partial(
        pl.kernel,
        mesh=mesh,
        out_type=jax.ShapeDtypeStruct((B, DIM), jnp.float32),
        compiler_params=pltpu.CompilerParams(needs_layout_passes=False),
        scratch_types=[
            pltpu.VMEM((CHUNKS + 1, G), jnp.int32),
            pltpu.VMEM((RPW, LPAD), jnp.float32),
            pltpu.VMEM((RPW, LPAD), jnp.int32),
            pltpu.VMEM((G, 4 * DIM), jnp.float32),
            pltpu.VMEM((G, 4 * DIM), jnp.float32),
            pltpu.VMEM((RPW, DIM), jnp.float32),
            pltpu.SemaphoreType.DMA,
            pltpu.SemaphoreType.DMA,
        ],
    )
    def emb(idx_hbm, w_hbm, off_hbm, table_hbm, out_hbm,
            idx_v, w_v, off_v, buf_a, buf_b, out_v, sem_a, sem_b):
        wid = lax.axis_index("s") * NC + lax.axis_index("c")
        pltpu.sync_copy(idx_hbm.at[wid], idx_v)
        pltpu.sync_copy(w_hbm.at[wid], w_v)
        pltpu.sync_copy(off_hbm.at[wid], off_v)

        def start(jj, buf, sem):
            pltpu.async_copy(table_hbm.at[idx_v.at[jj]], buf, sem)

        def wait(jj, buf, sem):
            pltpu.make_async_copy(table_hbm.at[idx_v.at[jj]], buf, sem).wait()

        def compute(buf, jj):
            for r in range(ROWS_PER_CHUNK):
                b = ROWS_PER_CHUNK * jj + r
                wblk = [w_v[b, pl.ds(k * LANES, LANES)] for k in range(4)]
                oblk = [off_v[b, pl.ds(k * LANES, LANES)] for k in range(4)]
                dv = wblk[0] + wblk[1] + wblk[2] + wblk[3]
                inv = 1.0 / _allsum(dv)
                a0 = jnp.zeros((LANES,), jnp.float32)
                a1 = jnp.zeros((LANES,), jnp.float32)
                for l in range(L):
                    w = _bcast(wblk[l // LANES], l % LANES)
                    o = oblk[l // LANES][l % LANES]
                    pos = r * L + l
                    a0 = a0 + w * buf[pos, pl.ds(o, LANES)]
                    a1 = a1 + w * buf[pos, pl.ds(o + LANES, LANES)]
                out_v[b, pl.ds(0, LANES)] = a0 * inv
                out_v[b, pl.ds(LANES, LANES)] = a1 * inv

        start(0, buf_a, sem_a)

        def body(j2, carry):
            jj = 2 * j2
            wait(jj, buf_a, sem_a)
            start(jj + 1, buf_b, sem_b)
            compute(buf_a, jj)
            wait(jj + 1, buf_b, sem_b)
            start(jj + 2, buf_a, sem_a)
            compute(buf_b, jj + 1)
            return carry

        lax.fori_loop(0, CHUNKS // 2, body, 0)
        wait(CHUNKS, buf_a, sem_a)
        pltpu.sync_copy(out_v, out_hbm.at[pl.ds(wid * RPW, RPW)])

    return emb


_tr_kernel = _make_transpose_kernel()
_emb_kernel = _make_emb_kernel()


def kernel(x_id, x_weight, word_vector):
    wv = word_vector.astype(jnp.float32)
    wvt = wv.T
    n_tail = VOCAB - FULL_COLS
    tail = jnp.pad(wv[FULL_COLS:], ((0, 128 - n_tail), (0, 0))).T
    table = _tr_kernel(wvt, tail)

    ids = x_id.astype(jnp.int32)
    idx = (ids // PACK).reshape(NW, CHUNKS, G)
    idx = jnp.concatenate([idx, idx[:, :1]], axis=1)
    off = (ids % PACK) * DIM
    off = jnp.pad(off, ((0, 0), (0, LPAD - L))).reshape(NW, RPW, LPAD)
    w = jnp.pad(x_weight.astype(jnp.float32), ((0, 0), (0, LPAD - L)))
    w = w.reshape(NW, RPW, LPAD)
    return _emb_kernel(idx, w, off, table)
